# baseline (speedup 1.0000x reference)
import functools

import jax
import jax.numpy as jnp
from jax import lax
from jax.experimental import pallas as pl
from jax.experimental.pallas import tpu as pltpu
from jax.experimental.pallas import tpu_sc as plsc

N = 10000
E = 320000
D = 128
NUM_LAYERS = 2

NC = 2
NS = 16
NW = NC * NS
CH = 128
NB = 8
E_PAD = 327680
CHUNKS = E_PAD // CH
EPW = E_PAD // NW
CPW = EPW // CH
N_PAD = 10240
ROWS_PER_SUB = N_PAD // NS

_mesh = plsc.VectorSubcoreMesh(core_axis_name="c", subcore_axis_name="s")


@functools.partial(
    pl.kernel,
    mesh=_mesh,
    out_type=jax.ShapeDtypeStruct((NC, N_PAD, D), jnp.float32),
    scratch_types=[
        pltpu.VMEM((16, CH), jnp.int32),
        pltpu.VMEM((16, CH), jnp.int32),
        pltpu.VMEM((16, CH), jnp.int32),
        pltpu.VMEM((16, CH), jnp.int32),
        pltpu.VMEM((CH, D), jnp.float32),
        pltpu.VMEM((CH, D), jnp.float32),
        pltpu.VMEM_SHARED((N_PAD, D), jnp.float32),
        pltpu.SemaphoreType.DMA,
        pltpu.SemaphoreType.DMA,
        pltpu.SemaphoreType.DMA,
        pltpu.SemaphoreType.DMA,
    ],
)
def _sc_segment_sum(m_hbm, src_hbm, dst_hbm, zero_hbm, out_hbm,
                    srca, dsta, srcb, dstb, rows0, rows1, agg_sh,
                    semg0, sems0, sems1, semi):
    c = lax.axis_index("c")
    s = lax.axis_index("s")
    wid = s * NC + c
    row0 = s * ROWS_PER_SUB

    base = wid * CPW

    ia = pltpu.async_copy(src_hbm.at[pl.ds(base, 16)], srca, semi)
    ib = pltpu.async_copy(dst_hbm.at[pl.ds(base, 16)], dsta, semi)

    pltpu.sync_copy(zero_hbm, rows0)
    for t in range(ROWS_PER_SUB // CH):
        pltpu.sync_copy(rows0, agg_sh.at[pl.ds(row0 + t * CH, CH)])
    plsc.subcore_barrier()
    ia.wait()
    ib.wait()

    rows = (rows0, rows1)
    sems = (sems0, sems1)

    def run_group(src_v, dst_v, s_prev):
        for j in range(16):
            pltpu.async_copy(m_hbm.at[src_v.at[j]],
                             rows[j % 2], semg0).wait()
            if s_prev is not None:
                s_prev.wait()
            s_prev = pltpu.async_copy(rows[j % 2],
                                      agg_sh.at[dst_v.at[j]],
                                      sems[j % 2], add=True)
        return s_prev

    @pl.loop(0, CPW, step=32)
    def _(g):
        pa0 = pltpu.async_copy(src_hbm.at[pl.ds(base + g + 16, 16)], srcb,
                               semi)
        pa1 = pltpu.async_copy(dst_hbm.at[pl.ds(base + g + 16, 16)], dstb,
                               semi)
        s_prev = run_group(srca, dsta, None)
        pa0.wait()
        pa1.wait()
        nxt = jnp.minimum(base + g + 32, CHUNKS - 16)
        pb0 = pltpu.async_copy(src_hbm.at[pl.ds(nxt, 16)], srca, semi)
        pb1 = pltpu.async_copy(dst_hbm.at[pl.ds(nxt, 16)], dsta, semi)
        s_prev = run_group(srcb, dstb, s_prev)
        s_prev.wait()
        pb0.wait()
        pb1.wait()

    plsc.subcore_barrier()
    pltpu.sync_copy(agg_sh.at[pl.ds(row0, ROWS_PER_SUB)],
                    out_hbm.at[c, pl.ds(row0, ROWS_PER_SUB)])


BLK = 2000


def _dot(a, b):
    return jnp.dot(a, b, preferred_element_type=jnp.float32)


def _linear_body(h_ref, w_ref, b_ref, o_ref):
    o_ref[...] = _dot(h_ref[...], w_ref[...]) + b_ref[...]


def _tc_linear(h, w, b):
    return pl.pallas_call(
        _linear_body,
        grid=(N // BLK,),
        in_specs=[
            pl.BlockSpec((BLK, D), lambda i: (i, 0)),
            pl.BlockSpec((D, D), lambda i: (0, 0)),
            pl.BlockSpec((1, D), lambda i: (0, 0)),
        ],
        out_specs=pl.BlockSpec((BLK, D), lambda i: (i, 0)),
        out_shape=jax.ShapeDtypeStruct((N, D), jnp.float32),
    )(h, w, b.reshape(1, D))


def _gru_body(h_ref, agg_ref, Wz_ref, Uz_ref, bz_ref, Wr_ref, Ur_ref, br_ref,
              Wh_ref, Uh_ref, bh_ref, Wm_ref, bm_ref, h_out_ref, m_out_ref):
    agg = agg_ref[0] + agg_ref[1]
    h = h_ref[...]
    z = jax.nn.sigmoid(_dot(agg, Wz_ref[...]) + _dot(h, Uz_ref[...])
                       + bz_ref[...])
    r = jax.nn.sigmoid(_dot(agg, Wr_ref[...]) + _dot(h, Ur_ref[...])
                       + br_ref[...])
    h_t = jnp.tanh(_dot(agg, Wh_ref[...]) + _dot(r * h, Uh_ref[...])
                   + bh_ref[...])
    h_new = (1.0 - z) * h + z * h_t
    h_out_ref[...] = h_new
    if m_out_ref is not None:
        m_out_ref[...] = _dot(h_new, Wm_ref[...]) + bm_ref[...]


def _tc_gru(h, agg2, Wz, Uz, bz, Wr, Ur, br, Wh, Uh, bh, Wm, bm,
            compute_m):
    n_out = 2 if compute_m else 1
    body = _gru_body if compute_m else (
        lambda *refs: _gru_body(*refs, None))
    wspec = pl.BlockSpec((D, D), lambda i: (0, 0))
    bspec = pl.BlockSpec((1, D), lambda i: (0, 0))
    rowspec = pl.BlockSpec((BLK, D), lambda i: (i, 0))
    out = pl.pallas_call(
        body,
        grid=(N // BLK,),
        in_specs=[
            rowspec,
            pl.BlockSpec((NC, BLK, D), lambda i: (0, i, 0)),
            wspec, wspec, bspec,
            wspec, wspec, bspec,
            wspec, wspec, bspec,
            wspec, bspec,
        ],
        out_specs=[rowspec] * n_out,
        out_shape=[jax.ShapeDtypeStruct((N, D), jnp.float32)] * n_out,
    )(h, agg2, Wz, Uz, bz.reshape(1, D), Wr, Ur, br.reshape(1, D),
      Wh, Uh, bh.reshape(1, D), Wm, bm.reshape(1, D))
    return out


@jax.jit
def kernel(features, edge_index, W_msg, b_msg, Wz, Uz, bz, Wr, Ur, br,
           Wh, Uh, bh):
    pad = E_PAD - E
    src_pad = jnp.arange(pad, dtype=jnp.int32) % N
    src = jnp.concatenate([edge_index[0], src_pad]).reshape(CHUNKS, CH)
    dst_pad = N + (jnp.arange(pad, dtype=jnp.int32) % (N_PAD - N))
    dst = jnp.concatenate([edge_index[1], dst_pad]).reshape(CHUNKS, CH)
    zeros = jnp.zeros((CH, D), jnp.float32)
    h = features
    m = _tc_linear(features, W_msg, b_msg)
    for layer in range(NUM_LAYERS):
        agg2 = _sc_segment_sum(m, src, dst, zeros)
        compute_m = layer < NUM_LAYERS - 1
        out = _tc_gru(h, agg2, Wz, Uz, bz, Wr, Ur, br, Wh, Uh, bh,
                      W_msg, b_msg, compute_m)
        if compute_m:
            h, m = out
        else:
            (h,) = out
    return h


# speedup vs baseline: 1.1395x; 1.1395x over previous
import functools

import jax
import jax.numpy as jnp
from jax import lax
from jax.experimental import pallas as pl
from jax.experimental.pallas import tpu as pltpu
from jax.experimental.pallas import tpu_sc as plsc

N = 10000
E = 320000
D = 128
NUM_LAYERS = 2

NC = 2
NS = 16
NW = NC * NS
CH = 128
NB = 8
E_PAD = 327680
CHUNKS = E_PAD // CH
EPW = E_PAD // NW
CPW = EPW // CH
N_PAD = 10240
ROWS_PER_SUB = N_PAD // NS

_mesh = plsc.VectorSubcoreMesh(core_axis_name="c", subcore_axis_name="s")


@functools.partial(
    pl.kernel,
    mesh=_mesh,
    out_type=jax.ShapeDtypeStruct((NC, N_PAD, D), jnp.float32),
    scratch_types=[
        pltpu.VMEM((8, CH), jnp.int32),
        pltpu.VMEM((8, CH), jnp.int32),
        pltpu.VMEM((8, CH), jnp.int32),
        pltpu.VMEM((8, CH), jnp.int32),
        pltpu.VMEM((CH, D), jnp.float32),
        pltpu.VMEM((CH, D), jnp.float32),
        pltpu.VMEM_SHARED((N_PAD, D), jnp.float32),
        pltpu.SemaphoreType.DMA,
        pltpu.SemaphoreType.DMA,
        pltpu.SemaphoreType.DMA,
        pltpu.SemaphoreType.DMA,
    ],
)
def _sc_segment_sum(m_hbm, src_hbm, dst_hbm, zero_hbm, out_hbm,
                    srca, dsta, srcb, dstb, rows0, rows1, agg_sh,
                    semg0, sems0, sems1, semi):
    c = lax.axis_index("c")
    s = lax.axis_index("s")
    wid = s * NC + c
    row0 = s * ROWS_PER_SUB

    base = wid * CPW

    ia = pltpu.async_copy(src_hbm.at[pl.ds(base, 8)], srca, semi)
    ib = pltpu.async_copy(dst_hbm.at[pl.ds(base, 8)], dsta, semi)

    pltpu.sync_copy(zero_hbm, rows0)
    for t in range(ROWS_PER_SUB // CH):
        pltpu.sync_copy(rows0, agg_sh.at[pl.ds(row0 + t * CH, CH)])
    plsc.subcore_barrier()
    ia.wait()
    ib.wait()

    rows = (rows0, rows1)
    sems = (sems0, sems1)

    def run_group(src_v, dst_v, s_prev):
        for j in range(8):
            pltpu.async_copy(m_hbm.at[src_v.at[j]],
                             rows[j % 2], semg0).wait()
            if s_prev is not None:
                s_prev.wait()
            s_prev = pltpu.async_copy(rows[j % 2],
                                      agg_sh.at[dst_v.at[j]],
                                      sems[j % 2], add=True)
        return s_prev

    @pl.loop(0, CPW, step=16)
    def _(g):
        pa0 = pltpu.async_copy(src_hbm.at[pl.ds(base + g + 8, 8)], srcb,
                               semi)
        pa1 = pltpu.async_copy(dst_hbm.at[pl.ds(base + g + 8, 8)], dstb,
                               semi)
        s_prev = run_group(srca, dsta, None)
        pa0.wait()
        pa1.wait()
        nxt = jnp.minimum(base + g + 16, CHUNKS - 8)
        pb0 = pltpu.async_copy(src_hbm.at[pl.ds(nxt, 8)], srca, semi)
        pb1 = pltpu.async_copy(dst_hbm.at[pl.ds(nxt, 8)], dsta, semi)
        s_prev = run_group(srcb, dstb, s_prev)
        s_prev.wait()
        pb0.wait()
        pb1.wait()

    plsc.subcore_barrier()
    pltpu.sync_copy(agg_sh.at[pl.ds(row0, ROWS_PER_SUB)],
                    out_hbm.at[c, pl.ds(row0, ROWS_PER_SUB)])


BLK = 2000


def _dot(a, b):
    return jnp.dot(a, b, preferred_element_type=jnp.float32)


def _linear_body(h_ref, w_ref, b_ref, o_ref):
    o_ref[...] = _dot(h_ref[...], w_ref[...]) + b_ref[...]


def _tc_linear(h, w, b):
    return pl.pallas_call(
        _linear_body,
        grid=(N // BLK,),
        in_specs=[
            pl.BlockSpec((BLK, D), lambda i: (i, 0)),
            pl.BlockSpec((D, D), lambda i: (0, 0)),
            pl.BlockSpec((1, D), lambda i: (0, 0)),
        ],
        out_specs=pl.BlockSpec((BLK, D), lambda i: (i, 0)),
        out_shape=jax.ShapeDtypeStruct((N, D), jnp.float32),
    )(h, w, b.reshape(1, D))


def _gru_body(h_ref, agg_ref, Wz_ref, Uz_ref, bz_ref, Wr_ref, Ur_ref, br_ref,
              Wh_ref, Uh_ref, bh_ref, Wm_ref, bm_ref, h_out_ref, m_out_ref):
    agg = agg_ref[0] + agg_ref[1]
    h = h_ref[...]
    z = jax.nn.sigmoid(_dot(agg, Wz_ref[...]) + _dot(h, Uz_ref[...])
                       + bz_ref[...])
    r = jax.nn.sigmoid(_dot(agg, Wr_ref[...]) + _dot(h, Ur_ref[...])
                       + br_ref[...])
    h_t = jnp.tanh(_dot(agg, Wh_ref[...]) + _dot(r * h, Uh_ref[...])
                   + bh_ref[...])
    h_new = (1.0 - z) * h + z * h_t
    h_out_ref[...] = h_new
    if m_out_ref is not None:
        m_out_ref[...] = _dot(h_new, Wm_ref[...]) + bm_ref[...]


def _tc_gru(h, agg2, Wz, Uz, bz, Wr, Ur, br, Wh, Uh, bh, Wm, bm,
            compute_m):
    n_out = 2 if compute_m else 1
    body = _gru_body if compute_m else (
        lambda *refs: _gru_body(*refs, None))
    wspec = pl.BlockSpec((D, D), lambda i: (0, 0))
    bspec = pl.BlockSpec((1, D), lambda i: (0, 0))
    rowspec = pl.BlockSpec((BLK, D), lambda i: (i, 0))
    out = pl.pallas_call(
        body,
        grid=(N // BLK,),
        in_specs=[
            rowspec,
            pl.BlockSpec((NC, BLK, D), lambda i: (0, i, 0)),
            wspec, wspec, bspec,
            wspec, wspec, bspec,
            wspec, wspec, bspec,
            wspec, bspec,
        ],
        out_specs=[rowspec] * n_out,
        out_shape=[jax.ShapeDtypeStruct((N, D), jnp.float32)] * n_out,
    )(h, agg2, Wz, Uz, bz.reshape(1, D), Wr, Ur, br.reshape(1, D),
      Wh, Uh, bh.reshape(1, D), Wm, bm.reshape(1, D))
    return out


@jax.jit
def kernel(features, edge_index, W_msg, b_msg, Wz, Uz, bz, Wr, Ur, br,
           Wh, Uh, bh):
    pad = E_PAD - E
    src_pad = jnp.arange(pad, dtype=jnp.int32) % N
    src = jnp.concatenate([edge_index[0], src_pad]).reshape(CHUNKS, CH)
    dst_pad = N + (jnp.arange(pad, dtype=jnp.int32) % (N_PAD - N))
    dst = jnp.concatenate([edge_index[1], dst_pad]).reshape(CHUNKS, CH)
    zeros = jnp.zeros((CH, D), jnp.float32)
    h = features
    m = _tc_linear(features, W_msg, b_msg)
    for layer in range(NUM_LAYERS):
        agg2 = _sc_segment_sum(m, src, dst, zeros)
        compute_m = layer < NUM_LAYERS - 1
        out = _tc_gru(h, agg2, Wz, Uz, bz, Wr, Ur, br, Wh, Uh, bh,
                      W_msg, b_msg, compute_m)
        if compute_m:
            h, m = out
        else:
            (h,) = out
    return h


# speedup vs baseline: 1.1397x; 1.0002x over previous
import functools

import jax
import jax.numpy as jnp
from jax import lax
from jax.experimental import pallas as pl
from jax.experimental.pallas import tpu as pltpu
from jax.experimental.pallas import tpu_sc as plsc

N = 10000
E = 320000
D = 128
NUM_LAYERS = 2

NC = 2
NS = 16
NW = NC * NS
CH = 128
NB = 8
E_PAD = 327680
CHUNKS = E_PAD // CH
EPW = E_PAD // NW
CPW = EPW // CH
N_PAD = 10240
ROWS_PER_SUB = N_PAD // NS

_mesh = plsc.VectorSubcoreMesh(core_axis_name="c", subcore_axis_name="s")


@functools.partial(
    pl.kernel,
    mesh=_mesh,
    out_type=jax.ShapeDtypeStruct((NC, N_PAD, D), jnp.float32),
    scratch_types=[
        pltpu.VMEM((8, CH), jnp.int32),
        pltpu.VMEM((8, CH), jnp.int32),
        pltpu.VMEM((8, CH), jnp.int32),
        pltpu.VMEM((8, CH), jnp.int32),
        pltpu.VMEM((CH, D), jnp.float32),
        pltpu.VMEM((CH, D), jnp.float32),
        pltpu.VMEM_SHARED((N_PAD, D), jnp.float32),
        pltpu.SemaphoreType.DMA,
        pltpu.SemaphoreType.DMA,
        pltpu.SemaphoreType.DMA,
        pltpu.SemaphoreType.DMA,
    ],
)
def _sc_segment_sum(m_hbm, src_hbm, dst_hbm, zero_hbm, out_hbm,
                    srca, dsta, srcb, dstb, rows0, rows1, agg_sh,
                    semg0, sems0, sems1, semi):
    c = lax.axis_index("c")
    s = lax.axis_index("s")
    wid = s * NC + c
    row0 = s * ROWS_PER_SUB

    base = wid * CPW

    ia = pltpu.async_copy(src_hbm.at[pl.ds(base, 8)], srca, semi)
    ib = pltpu.async_copy(dst_hbm.at[pl.ds(base, 8)], dsta, semi)

    pltpu.sync_copy(zero_hbm, rows0)
    for t in range(ROWS_PER_SUB // CH):
        pltpu.sync_copy(rows0, agg_sh.at[pl.ds(row0 + t * CH, CH)])
    plsc.subcore_barrier()
    ia.wait()
    ib.wait()

    rows = (rows0, rows1)
    sems = (sems0, sems1)

    def run_group(src_v, dst_v, s_state):
        for j in range(8):
            if s_state[j % 2] is not None:
                s_state[j % 2].wait()
            pltpu.async_copy(m_hbm.at[src_v.at[j]],
                             rows[j % 2], semg0).wait()
            s_state[j % 2] = pltpu.async_copy(rows[j % 2],
                                              agg_sh.at[dst_v.at[j]],
                                              sems[j % 2], add=True)
        return s_state

    @pl.loop(0, CPW, step=16)
    def _(g):
        pa0 = pltpu.async_copy(src_hbm.at[pl.ds(base + g + 8, 8)], srcb,
                               semi)
        pa1 = pltpu.async_copy(dst_hbm.at[pl.ds(base + g + 8, 8)], dstb,
                               semi)
        s_state = run_group(srca, dsta, [None, None])
        pa0.wait()
        pa1.wait()
        nxt = jnp.minimum(base + g + 16, CHUNKS - 8)
        pb0 = pltpu.async_copy(src_hbm.at[pl.ds(nxt, 8)], srca, semi)
        pb1 = pltpu.async_copy(dst_hbm.at[pl.ds(nxt, 8)], dsta, semi)
        s_state = run_group(srcb, dstb, s_state)
        s_state[0].wait()
        s_state[1].wait()
        pb0.wait()
        pb1.wait()

    plsc.subcore_barrier()
    pltpu.sync_copy(agg_sh.at[pl.ds(row0, ROWS_PER_SUB)],
                    out_hbm.at[c, pl.ds(row0, ROWS_PER_SUB)])


BLK = 2000


def _dot(a, b):
    return jnp.dot(a, b, preferred_element_type=jnp.float32)


def _linear_body(h_ref, w_ref, b_ref, o_ref):
    o_ref[...] = _dot(h_ref[...], w_ref[...]) + b_ref[...]


def _tc_linear(h, w, b):
    return pl.pallas_call(
        _linear_body,
        grid=(N // BLK,),
        in_specs=[
            pl.BlockSpec((BLK, D), lambda i: (i, 0)),
            pl.BlockSpec((D, D), lambda i: (0, 0)),
            pl.BlockSpec((1, D), lambda i: (0, 0)),
        ],
        out_specs=pl.BlockSpec((BLK, D), lambda i: (i, 0)),
        out_shape=jax.ShapeDtypeStruct((N, D), jnp.float32),
    )(h, w, b.reshape(1, D))


def _gru_body(h_ref, agg_ref, Wz_ref, Uz_ref, bz_ref, Wr_ref, Ur_ref, br_ref,
              Wh_ref, Uh_ref, bh_ref, Wm_ref, bm_ref, h_out_ref, m_out_ref):
    agg = agg_ref[0] + agg_ref[1]
    h = h_ref[...]
    z = jax.nn.sigmoid(_dot(agg, Wz_ref[...]) + _dot(h, Uz_ref[...])
                       + bz_ref[...])
    r = jax.nn.sigmoid(_dot(agg, Wr_ref[...]) + _dot(h, Ur_ref[...])
                       + br_ref[...])
    h_t = jnp.tanh(_dot(agg, Wh_ref[...]) + _dot(r * h, Uh_ref[...])
                   + bh_ref[...])
    h_new = (1.0 - z) * h + z * h_t
    h_out_ref[...] = h_new
    if m_out_ref is not None:
        m_out_ref[...] = _dot(h_new, Wm_ref[...]) + bm_ref[...]


def _tc_gru(h, agg2, Wz, Uz, bz, Wr, Ur, br, Wh, Uh, bh, Wm, bm,
            compute_m):
    n_out = 2 if compute_m else 1
    body = _gru_body if compute_m else (
        lambda *refs: _gru_body(*refs, None))
    wspec = pl.BlockSpec((D, D), lambda i: (0, 0))
    bspec = pl.BlockSpec((1, D), lambda i: (0, 0))
    rowspec = pl.BlockSpec((BLK, D), lambda i: (i, 0))
    out = pl.pallas_call(
        body,
        grid=(N // BLK,),
        in_specs=[
            rowspec,
            pl.BlockSpec((NC, BLK, D), lambda i: (0, i, 0)),
            wspec, wspec, bspec,
            wspec, wspec, bspec,
            wspec, wspec, bspec,
            wspec, bspec,
        ],
        out_specs=[rowspec] * n_out,
        out_shape=[jax.ShapeDtypeStruct((N, D), jnp.float32)] * n_out,
    )(h, agg2, Wz, Uz, bz.reshape(1, D), Wr, Ur, br.reshape(1, D),
      Wh, Uh, bh.reshape(1, D), Wm, bm.reshape(1, D))
    return out


@jax.jit
def kernel(features, edge_index, W_msg, b_msg, Wz, Uz, bz, Wr, Ur, br,
           Wh, Uh, bh):
    pad = E_PAD - E
    src_pad = jnp.arange(pad, dtype=jnp.int32) % N
    src = jnp.concatenate([edge_index[0], src_pad]).reshape(CHUNKS, CH)
    dst_pad = N + (jnp.arange(pad, dtype=jnp.int32) % (N_PAD - N))
    dst = jnp.concatenate([edge_index[1], dst_pad]).reshape(CHUNKS, CH)
    zeros = jnp.zeros((CH, D), jnp.float32)
    h = features
    m = _tc_linear(features, W_msg, b_msg)
    for layer in range(NUM_LAYERS):
        agg2 = _sc_segment_sum(m, src, dst, zeros)
        compute_m = layer < NUM_LAYERS - 1
        out = _tc_gru(h, agg2, Wz, Uz, bz, Wr, Ur, br, Wh, Uh, bh,
                      W_msg, b_msg, compute_m)
        if compute_m:
            h, m = out
        else:
            (h,) = out
    return h
